# Initial kernel scaffold; baseline (speedup 1.0000x reference)
import jax
import jax.numpy as jnp
from jax.experimental import pallas as pl


def kernel(x, edge_index, W1, b1, W2, b2):
    raise NotImplementedError("write your pallas kernel here")


# speedup vs baseline: 19.4306x; 19.4306x over previous
import functools

import jax
import jax.numpy as jnp
from jax import lax
from jax.experimental import pallas as pl
from jax.experimental.pallas import tpu as pltpu
from jax.experimental.pallas import tpu_sc as plsc

N = 10000
F = 256
HID = 256
CLS = 64
E = 160000

NC = 2
NS = 16
LANES = 16

EDGES_PER_SUB = E // NS
CHUNK = 80
NCHUNK = EDGES_PER_SUB // CHUNK
NBUF = 5
N_PAD = 10112
ROWS_PER_SUB = N_PAD // NS

DEG_PER_W = E // (NC * NS)
_DEG_FULL = DEG_PER_W // LANES
_DEG_TAIL = DEG_PER_W - _DEG_FULL * LANES

_sc_mesh = plsc.VectorSubcoreMesh(core_axis_name="c", subcore_axis_name="s")


@functools.partial(
    pl.kernel,
    out_type=jax.ShapeDtypeStruct((NC * NS, N), jnp.float32),
    mesh=_sc_mesh,
    scratch_types=[
        pltpu.VMEM((DEG_PER_W + LANES,), jnp.int32),
        pltpu.VMEM((N,), jnp.float32),
    ],
    compiler_params=pltpu.CompilerParams(needs_layout_passes=False,
                                         use_tc_tiling_on_sc=False),
)
def _deg_kernel(dst_hbm, out_hbm, idx_v, hist_v):
    c = lax.axis_index("c")
    s = lax.axis_index("s")
    wid = s * NC + c
    base = wid * DEG_PER_W
    idx_v[pl.ds(_DEG_FULL * LANES, LANES)] = jnp.zeros((LANES,), jnp.int32)
    pltpu.sync_copy(dst_hbm.at[pl.ds(base, DEG_PER_W)],
                    idx_v.at[pl.ds(0, DEG_PER_W)])

    @pl.loop(0, N // LANES)
    def _zero(i):
        hist_v[pl.ds(i * LANES, LANES)] = jnp.zeros((LANES,), jnp.float32)

    ones = jnp.ones((LANES,), jnp.float32)

    @pl.loop(0, _DEG_FULL)
    def _acc(i):
        idx = idx_v[pl.ds(i * LANES, LANES)]
        plsc.addupdate_scatter(hist_v, [idx], ones)

    tidx = idx_v[pl.ds(_DEG_FULL * LANES, LANES)]
    tmask = lax.iota(jnp.int32, LANES) < _DEG_TAIL
    plsc.addupdate_scatter(hist_v, [tidx], ones, mask=tmask)
    pltpu.sync_copy(hist_v, out_hbm.at[wid])


def _make_scatter(width):
    @functools.partial(
        pl.kernel,
        out_type=jax.ShapeDtypeStruct((NC, N_PAD, width), jnp.float32),
        mesh=_sc_mesh,
        scratch_types=[
            pltpu.VMEM((NCHUNK, CHUNK), jnp.int32),
            pltpu.VMEM((NCHUNK, CHUNK), jnp.int32),
            pltpu.VMEM((NBUF, CHUNK, width), jnp.float32),
            pltpu.VMEM_SHARED((N_PAD, width), jnp.float32),
            pltpu.SemaphoreType.DMA((NBUF,)),
        ],
        compiler_params=pltpu.CompilerParams(needs_layout_passes=False,
                                             use_tc_tiling_on_sc=False),
    )
    def _scatter(hs_hbm, srco_hbm, dst_hbm, zeros_hbm, out_hbm,
                 src_v, dst_v, rows_v, acc, sems):
        c = lax.axis_index("c")
        s = lax.axis_index("s")
        pltpu.sync_copy(srco_hbm.at[c, s], src_v)
        pltpu.sync_copy(dst_hbm.at[s], dst_v)
        pltpu.sync_copy(zeros_hbm.at[pl.ds(s * ROWS_PER_SUB, ROWS_PER_SUB)],
                        acc.at[pl.ds(s * ROWS_PER_SUB, ROWS_PER_SUB)])
        plsc.subcore_barrier()

        for b in range(NBUF):
            pltpu.async_copy(hs_hbm.at[src_v.at[b]], rows_v.at[b], sems.at[b])

        @pl.loop(0, NCHUNK // NBUF - 1)
        def _grp(g):
            for b in range(NBUF):
                k = g * NBUF + b
                pltpu.make_async_copy(hs_hbm.at[src_v.at[k]], rows_v.at[b],
                                      sems.at[b]).wait()
                pltpu.sync_copy(rows_v.at[b], acc.at[dst_v.at[k]], add=True)
                pltpu.async_copy(hs_hbm.at[src_v.at[k + NBUF]], rows_v.at[b],
                                 sems.at[b])

        for b in range(NBUF):
            k = NCHUNK - NBUF + b
            pltpu.make_async_copy(hs_hbm.at[src_v.at[k]], rows_v.at[b],
                                  sems.at[b]).wait()
            pltpu.sync_copy(rows_v.at[b], acc.at[dst_v.at[k]], add=True)

        plsc.subcore_barrier()
        pltpu.sync_copy(acc.at[pl.ds(s * ROWS_PER_SUB, ROWS_PER_SUB)],
                        out_hbm.at[c, pl.ds(s * ROWS_PER_SUB, ROWS_PER_SUB)])

    return _scatter


_scatter_hid = _make_scatter(HID // 4)
_scatter_cls = _make_scatter(CLS // 2)


_NB = 1000
_GRID = N // _NB


def _mm1_body(x_ref, w_ref, o_ref):
    o_ref[...] = jnp.dot(x_ref[...], w_ref[...],
                         preferred_element_type=jnp.float32)


_mm1 = pl.pallas_call(
    _mm1_body,
    grid=(_GRID,),
    in_specs=[pl.BlockSpec((_NB, F), lambda i: (i, 0)),
              pl.BlockSpec((F, HID), lambda i: (0, 0))],
    out_specs=pl.BlockSpec((_NB, HID), lambda i: (i, 0)),
    out_shape=jax.ShapeDtypeStruct((N, HID), jnp.float32),
)


def _dinv_of(dp_block):
    deg = jnp.sum(dp_block, axis=1) + 1.0
    return lax.rsqrt(deg)


def _prep_body(dp_ref, h_ref, oa_ref, ob_ref):
    dinv = _dinv_of(dp_ref[...])
    hs = h_ref[...] * dinv[:, None]
    q = HID // 4
    oa_ref[0] = hs[:, 0 * q:1 * q]
    oa_ref[1] = hs[:, 1 * q:2 * q]
    ob_ref[0] = hs[:, 2 * q:3 * q]
    ob_ref[1] = hs[:, 3 * q:4 * q]


_prep = pl.pallas_call(
    _prep_body,
    grid=(_GRID,),
    in_specs=[pl.BlockSpec((_NB, NC * NS), lambda i: (i, 0)),
              pl.BlockSpec((_NB, HID), lambda i: (i, 0))],
    out_specs=[pl.BlockSpec((2, _NB, HID // 4), lambda i: (0, i, 0)),
               pl.BlockSpec((2, _NB, HID // 4), lambda i: (0, i, 0))],
    out_shape=[jax.ShapeDtypeStruct((2, N, HID // 4), jnp.float32),
               jax.ShapeDtypeStruct((2, N, HID // 4), jnp.float32)],
)


def _mid_body(a_ref, a2_ref, h_ref, dp_ref, b_ref, w_ref, hs_ref, h2_ref):
    dinv = _dinv_of(dp_ref[...])
    agg = jnp.concatenate([a_ref[0], a_ref[1], a2_ref[0], a2_ref[1]], axis=1)
    z = (agg * dinv[:, None] + h_ref[...] * (dinv * dinv)[:, None]
         + b_ref[...])
    a = jnp.maximum(z, 0.0)
    h2 = jnp.dot(a, w_ref[...], preferred_element_type=jnp.float32)
    h2_ref[...] = h2
    hs2 = h2 * dinv[:, None]
    hs_ref[0] = hs2[:, :CLS // 2]
    hs_ref[1] = hs2[:, CLS // 2:]


_mid = pl.pallas_call(
    _mid_body,
    grid=(_GRID,),
    in_specs=[pl.BlockSpec((2, _NB, HID // 4), lambda i: (0, i, 0)),
              pl.BlockSpec((2, _NB, HID // 4), lambda i: (0, i, 0)),
              pl.BlockSpec((_NB, HID), lambda i: (i, 0)),
              pl.BlockSpec((_NB, NC * NS), lambda i: (i, 0)),
              pl.BlockSpec((1, HID), lambda i: (0, 0)),
              pl.BlockSpec((HID, CLS), lambda i: (0, 0))],
    out_specs=[pl.BlockSpec((2, _NB, CLS // 2), lambda i: (0, i, 0)),
               pl.BlockSpec((_NB, CLS), lambda i: (i, 0))],
    out_shape=[jax.ShapeDtypeStruct((2, N, CLS // 2), jnp.float32),
               jax.ShapeDtypeStruct((N, CLS), jnp.float32)],
)


def _final_body(a_ref, h2_ref, dp_ref, b_ref, o_ref):
    dinv = _dinv_of(dp_ref[...])
    agg = jnp.concatenate([a_ref[0], a_ref[1]], axis=1)
    z = (agg * dinv[:, None] + h2_ref[...] * (dinv * dinv)[:, None]
         + b_ref[...])
    m = jnp.max(z, axis=1, keepdims=True)
    lse = jnp.log(jnp.sum(jnp.exp(z - m), axis=1, keepdims=True)) + m
    o_ref[...] = z - lse


_final = pl.pallas_call(
    _final_body,
    grid=(_GRID,),
    in_specs=[pl.BlockSpec((2, _NB, CLS // 2), lambda i: (0, i, 0)),
              pl.BlockSpec((_NB, CLS), lambda i: (i, 0)),
              pl.BlockSpec((_NB, NC * NS), lambda i: (i, 0)),
              pl.BlockSpec((1, CLS), lambda i: (0, 0))],
    out_specs=pl.BlockSpec((_NB, CLS), lambda i: (i, 0)),
    out_shape=jax.ShapeDtypeStruct((N, CLS), jnp.float32),
)


def kernel(x, edge_index, W1, b1, W2, b2):
    src = edge_index[0].astype(jnp.int32)
    dst = edge_index[1].astype(jnp.int32)

    deg_parts = _deg_kernel(dst).T
    h1 = _mm1(x, W1)
    hs1a, hs1b = _prep(deg_parts, h1)

    srco = jnp.stack([src, src + N]).reshape(NC, NS, NCHUNK, CHUNK)
    dst3 = dst.reshape(NS, NCHUNK, CHUNK)

    z64 = jnp.zeros((N_PAD, HID // 4), jnp.float32)
    agg1a = _scatter_hid(hs1a.reshape(2 * N, HID // 4), srco, dst3, z64)
    agg1b = _scatter_hid(hs1b.reshape(2 * N, HID // 4), srco, dst3, z64)
    hs2, h2 = _mid(agg1a, agg1b, h1, deg_parts, b1.reshape(1, HID), W2)
    agg2 = _scatter_cls(hs2.reshape(2 * N, CLS // 2), srco, dst3,
                        jnp.zeros((N_PAD, CLS // 2), jnp.float32))
    return _final(agg2, h2, deg_parts, b2.reshape(1, CLS))
